# baseline (speedup 1.0000x reference)
import functools

import jax
import jax.numpy as jnp
from jax import lax
from jax.experimental import pallas as pl
from jax.experimental.pallas import tpu as pltpu
from jax.experimental.pallas import tpu_sc as plsc

_NUM_CORES = 2
_NUM_SUBCORES = 16
_NW = _NUM_CORES * _NUM_SUBCORES
_CHUNK = 64


@jax.jit
def _gather(table, x):
    b, t = x.shape
    n = b * t
    d = table.shape[1]
    per_w = n // _NW
    n_chunks = per_w // _CHUNK
    w_per_row = t // per_w

    mesh = plsc.VectorSubcoreMesh(core_axis_name="c", subcore_axis_name="s")

    @functools.partial(
        pl.kernel,
        mesh=mesh,
        out_type=jax.ShapeDtypeStruct((n, d), jnp.float32),
        scratch_types=[
            pltpu.VMEM((per_w,), jnp.int32),
            pltpu.VMEM((per_w, d), jnp.float32),
            pltpu.SemaphoreType.DMA,
            pltpu.SemaphoreType.DMA,
            pltpu.SemaphoreType.DMA,
        ],
    )
    def body(table_hbm, x_hbm, out_hbm, idx_v, rows_v, isem, gsem, wsem):
        wid = lax.axis_index("s") * _NUM_CORES + lax.axis_index("c")
        r = wid // w_per_row
        col = (wid % w_per_row) * per_w
        base = wid * per_w
        for j in range(n_chunks):
            pltpu.async_copy(
                x_hbm.at[r, pl.ds(col + j * _CHUNK, _CHUNK)],
                idx_v.at[pl.ds(j * _CHUNK, _CHUNK)],
                isem,
            )
        for j in range(n_chunks):
            pltpu.make_async_copy(
                x_hbm.at[r, pl.ds(col + j * _CHUNK, _CHUNK)],
                idx_v.at[pl.ds(j * _CHUNK, _CHUNK)],
                isem,
            ).wait()
            pltpu.async_copy(
                table_hbm.at[idx_v.at[pl.ds(j * _CHUNK, _CHUNK)]],
                rows_v.at[pl.ds(j * _CHUNK, _CHUNK)],
                gsem,
            )
        for j in range(n_chunks):
            pltpu.make_async_copy(
                table_hbm.at[idx_v.at[pl.ds(j * _CHUNK, _CHUNK)]],
                rows_v.at[pl.ds(j * _CHUNK, _CHUNK)],
                gsem,
            ).wait()
            pltpu.async_copy(
                rows_v.at[pl.ds(j * _CHUNK, _CHUNK)],
                out_hbm.at[pl.ds(base + j * _CHUNK, _CHUNK)],
                wsem,
            )
        for j in range(n_chunks):
            pltpu.make_async_copy(
                rows_v.at[pl.ds(j * _CHUNK, _CHUNK)],
                out_hbm.at[pl.ds(base + j * _CHUNK, _CHUNK)],
                wsem,
            ).wait()

    return body(table, x)


def kernel(x, token_embedding_table):
    b, t = x.shape
    d = token_embedding_table.shape[1]
    out = _gather(token_embedding_table, x)
    return out.reshape(b, t, d)


# speedup vs baseline: 1.0105x; 1.0105x over previous
import functools

import jax
import jax.numpy as jnp
from jax import lax
from jax.experimental import pallas as pl
from jax.experimental.pallas import tpu as pltpu
from jax.experimental.pallas import tpu_sc as plsc

_NUM_CORES = 2
_NUM_SUBCORES = 16
_NW = _NUM_CORES * _NUM_SUBCORES
_CHUNK = 128


@jax.jit
def _gather(table, x):
    b, t = x.shape
    n = b * t
    d = table.shape[1]
    per_w = n // _NW
    n_chunks = per_w // _CHUNK
    w_per_row = t // per_w

    mesh = plsc.VectorSubcoreMesh(core_axis_name="c", subcore_axis_name="s")

    @functools.partial(
        pl.kernel,
        mesh=mesh,
        out_type=jax.ShapeDtypeStruct((n, d), jnp.float32),
        scratch_types=[
            pltpu.VMEM((per_w,), jnp.int32),
            pltpu.VMEM((per_w, d), jnp.float32),
            pltpu.SemaphoreType.DMA,
            pltpu.SemaphoreType.DMA,
            pltpu.SemaphoreType.DMA,
        ],
    )
    def body(table_hbm, x_hbm, out_hbm, idx_v, rows_v, isem, gsem, wsem):
        wid = lax.axis_index("s") * _NUM_CORES + lax.axis_index("c")
        r = wid // w_per_row
        col = (wid % w_per_row) * per_w
        base = wid * per_w
        for j in range(n_chunks):
            pltpu.async_copy(
                x_hbm.at[r, pl.ds(col + j * _CHUNK, _CHUNK)],
                idx_v.at[pl.ds(j * _CHUNK, _CHUNK)],
                isem,
            )
        for j in range(n_chunks):
            pltpu.make_async_copy(
                x_hbm.at[r, pl.ds(col + j * _CHUNK, _CHUNK)],
                idx_v.at[pl.ds(j * _CHUNK, _CHUNK)],
                isem,
            ).wait()
            pltpu.async_copy(
                table_hbm.at[idx_v.at[pl.ds(j * _CHUNK, _CHUNK)]],
                rows_v.at[pl.ds(j * _CHUNK, _CHUNK)],
                gsem,
            )
        for j in range(n_chunks):
            pltpu.make_async_copy(
                table_hbm.at[idx_v.at[pl.ds(j * _CHUNK, _CHUNK)]],
                rows_v.at[pl.ds(j * _CHUNK, _CHUNK)],
                gsem,
            ).wait()
            pltpu.async_copy(
                rows_v.at[pl.ds(j * _CHUNK, _CHUNK)],
                out_hbm.at[pl.ds(base + j * _CHUNK, _CHUNK)],
                wsem,
            )
        for j in range(n_chunks):
            pltpu.make_async_copy(
                rows_v.at[pl.ds(j * _CHUNK, _CHUNK)],
                out_hbm.at[pl.ds(base + j * _CHUNK, _CHUNK)],
                wsem,
            ).wait()

    return body(table, x)


def kernel(x, token_embedding_table):
    b, t = x.shape
    d = token_embedding_table.shape[1]
    out = _gather(token_embedding_table, x)
    return out.reshape(b, t, d)
